# baseline (speedup 1.0000x reference)
import jax
import jax.numpy as jnp
from jax.experimental import pallas as pl
from jax.experimental.pallas import tpu as pltpu

_B = 1024
_S = 4096
_I = 256
_BJ = 512
_P = 2


def _esn_tile(inputs_ref, states_ref, win_ref, wres_ref, out_ref, sb_ref):
    jj = pl.program_id(1)

    @pl.when(jj == 0)
    def _cast_states():
        sb_ref[...] = states_ref[...].astype(jnp.bfloat16)

    wb = wres_ref[...].astype(jnp.bfloat16)
    ib = inputs_ref[...].astype(jnp.bfloat16)
    winb = win_ref[...].astype(jnp.bfloat16)
    z = jnp.dot(sb_ref[...], wb, preferred_element_type=jnp.float32)
    z = z + jnp.dot(ib, winb, preferred_element_type=jnp.float32)
    cand = jnp.tanh(z)
    j = pl.program_id(0) * pl.num_programs(1) + jj
    sj = states_ref[:, pl.ds(j * _BJ, _BJ)]
    out_ref[...] = sj + (cand - sj)


def kernel(inputs, states, Win, Wres):
    nj = _S // _BJ // _P
    grid = (_P, nj)
    return pl.pallas_call(
        _esn_tile,
        grid=grid,
        in_specs=[
            pl.BlockSpec((_B, _I), lambda p, j: (0, 0)),
            pl.BlockSpec((_B, _S), lambda p, j: (0, 0)),
            pl.BlockSpec((_I, _BJ), lambda p, j: (0, p * (_S // _BJ // _P) + j)),
            pl.BlockSpec((_S, _BJ), lambda p, j: (0, p * (_S // _BJ // _P) + j)),
        ],
        out_specs=pl.BlockSpec((_B, _BJ), lambda p, j: (0, p * (_S // _BJ // _P) + j)),
        out_shape=jax.ShapeDtypeStruct((_B, _S), jnp.float32),
        scratch_shapes=[pltpu.VMEM((_B, _S), jnp.bfloat16)],
        compiler_params=pltpu.CompilerParams(
            dimension_semantics=("parallel", "arbitrary"),
        ),
    )(inputs, states, Win, Wres)


# speedup vs baseline: 1.0321x; 1.0321x over previous
import jax
import jax.numpy as jnp
from jax.experimental import pallas as pl
from jax.experimental.pallas import tpu as pltpu

_B = 1024
_S = 4096
_I = 256
_BJ = 512
_P = 2


def _esn_tile(inputs_ref, states_ref, win_ref, wres_ref, out_ref, sb_ref):
    jj = pl.program_id(1)

    @pl.when(jj == 0)
    def _cast_states():
        sb_ref[...] = states_ref[...].astype(jnp.bfloat16)

    z = jnp.dot(states_ref[...], wres_ref[...],
                preferred_element_type=jnp.float32)
    z = z + jnp.dot(inputs_ref[...], win_ref[...],
                    preferred_element_type=jnp.float32)
    cand = jnp.tanh(z)
    j = pl.program_id(0) * pl.num_programs(1) + jj
    sj = states_ref[:, pl.ds(j * _BJ, _BJ)]
    out_ref[...] = sj + (cand - sj)


def kernel(inputs, states, Win, Wres):
    nj = _S // _BJ // _P
    grid = (_P, nj)
    return pl.pallas_call(
        _esn_tile,
        grid=grid,
        in_specs=[
            pl.BlockSpec((_B, _I), lambda p, j: (0, 0)),
            pl.BlockSpec((_B, _S), lambda p, j: (0, 0)),
            pl.BlockSpec((_I, _BJ), lambda p, j: (0, p * (_S // _BJ // _P) + j)),
            pl.BlockSpec((_S, _BJ), lambda p, j: (0, p * (_S // _BJ // _P) + j)),
        ],
        out_specs=pl.BlockSpec((_B, _BJ), lambda p, j: (0, p * (_S // _BJ // _P) + j)),
        out_shape=jax.ShapeDtypeStruct((_B, _S), jnp.float32),
        scratch_shapes=[pltpu.VMEM((_B, _S), jnp.bfloat16)],
        compiler_params=pltpu.CompilerParams(
            dimension_semantics=("parallel", "arbitrary"),
        ),
    )(inputs, states, Win, Wres)
